# initial kernel scaffold (unmeasured)
import jax
import jax.numpy as jnp
from jax import lax
from jax.experimental import pallas as pl
from jax.experimental.pallas import tpu as pltpu

N_DEV = 8
HOPS = N_DEV - 1


def kernel(x, w_mat):
    m, k_shard = x.shape
    _, n = w_mat.shape
    ch = m // N_DEV

    def body(x_ref, w_ref, out_ref, comm_ref, rs_send, rs_recv, ag_send, ag_recv):
        i = lax.axis_index("i")
        right = lax.rem(i + 1, N_DEV)

        out_ref[:, :] = jnp.dot(
            x_ref[:, :], w_ref[:, :], preferred_element_type=jnp.float32
        )

        for h in range(HOPS):
            c_send = lax.rem(i - h + N_DEV, N_DEV)
            c_recv = lax.rem(i - h - 1 + N_DEV, N_DEV)
            rdma = pltpu.make_async_remote_copy(
                src_ref=out_ref.at[pl.ds(c_send * ch, ch)],
                dst_ref=comm_ref.at[h],
                send_sem=rs_send.at[h],
                recv_sem=rs_recv.at[h],
                device_id=(right,),
                device_id_type=pl.DeviceIdType.MESH,
            )
            rdma.start()
            rdma.wait()
            out_ref[pl.ds(c_recv * ch, ch)] += comm_ref[h]

        for h in range(HOPS):
            c_send = lax.rem(i + 1 - h + N_DEV, N_DEV)
            rdma = pltpu.make_async_remote_copy(
                src_ref=out_ref.at[pl.ds(c_send * ch, ch)],
                dst_ref=out_ref.at[pl.ds(c_send * ch, ch)],
                send_sem=ag_send.at[h],
                recv_sem=ag_recv.at[h],
                device_id=(right,),
                device_id_type=pl.DeviceIdType.MESH,
            )
            rdma.start()
            rdma.wait()

    return pl.pallas_call(
        body,
        out_shape=jax.ShapeDtypeStruct((m, n), jnp.float32),
        in_specs=[
            pl.BlockSpec(memory_space=pltpu.VMEM),
            pl.BlockSpec(memory_space=pltpu.VMEM),
        ],
        out_specs=pl.BlockSpec(memory_space=pltpu.VMEM),
        scratch_shapes=[
            pltpu.VMEM((HOPS, ch, n), jnp.float32),
            pltpu.SemaphoreType.DMA((HOPS,)),
            pltpu.SemaphoreType.DMA((HOPS,)),
            pltpu.SemaphoreType.DMA((HOPS,)),
            pltpu.SemaphoreType.DMA((HOPS,)),
        ],
        compiler_params=pltpu.CompilerParams(collective_id=0),
    )(x, w_mat)


# baseline (device time: 360429 ns/iter reference)
import jax
import jax.numpy as jnp
from jax import lax
from jax.experimental import pallas as pl
from jax.experimental.pallas import tpu as pltpu

N_DEV = 8
HOPS = N_DEV - 1


def kernel(x, w_mat):
    m, k_shard = x.shape
    _, n = w_mat.shape
    ch = m // N_DEV

    def body(x_ref, w_ref, out_ref, comm_ref, rs_send, rs_recv, ag_send, ag_recv):
        i = lax.axis_index("i")
        right = lax.rem(i + 1, N_DEV)

        out_ref[:, :] = jnp.dot(
            x_ref[:, :], w_ref[:, :], preferred_element_type=jnp.float32
        )

        for h in range(HOPS):
            c_send = lax.rem(i - h + N_DEV, N_DEV)
            c_recv = lax.rem(i - h - 1 + N_DEV, N_DEV)
            rdma = pltpu.make_async_remote_copy(
                src_ref=out_ref.at[pl.ds(c_send * ch, ch)],
                dst_ref=comm_ref.at[h],
                send_sem=rs_send.at[h],
                recv_sem=rs_recv.at[h],
                device_id=(right,),
                device_id_type=pl.DeviceIdType.MESH,
            )
            rdma.start()
            rdma.wait()
            out_ref[pl.ds(c_recv * ch, ch)] += comm_ref[h]

        for h in range(HOPS):
            c_send = lax.rem(i + 1 - h + N_DEV, N_DEV)
            rdma = pltpu.make_async_remote_copy(
                src_ref=out_ref.at[pl.ds(c_send * ch, ch)],
                dst_ref=out_ref.at[pl.ds(c_send * ch, ch)],
                send_sem=ag_send.at[h],
                recv_sem=ag_recv.at[h],
                device_id=(right,),
                device_id_type=pl.DeviceIdType.MESH,
            )
            rdma.start()
            rdma.wait()

    return pl.pallas_call(
        body,
        out_shape=jax.ShapeDtypeStruct((m, n), jnp.float32),
        in_specs=[
            pl.BlockSpec(memory_space=pltpu.VMEM),
            pl.BlockSpec(memory_space=pltpu.VMEM),
        ],
        out_specs=pl.BlockSpec(memory_space=pltpu.VMEM),
        scratch_shapes=[
            pltpu.VMEM((HOPS, ch, n), jnp.float32),
            pltpu.SemaphoreType.DMA((HOPS,)),
            pltpu.SemaphoreType.DMA((HOPS,)),
            pltpu.SemaphoreType.DMA((HOPS,)),
            pltpu.SemaphoreType.DMA((HOPS,)),
        ],
    )(x, w_mat)


# device time: 207862 ns/iter; 1.7340x vs baseline; 1.7340x over previous
import jax
import jax.numpy as jnp
from jax import lax
from jax.experimental import pallas as pl
from jax.experimental.pallas import tpu as pltpu

N_DEV = 8
HOPS = N_DEV - 1


def kernel(x, w_mat):
    m, k_shard = x.shape
    _, n = w_mat.shape
    ch = m // N_DEV
    half = n // 2

    def body(
        x_ref, w_ref, out_ref,
        cw_ref, ccw_ref,
        cw_send, cw_recv, ccw_send, ccw_recv,
        ag_cw_send, ag_cw_recv, ag_ccw_send, ag_ccw_recv,
    ):
        i = lax.axis_index("i")
        right = lax.rem(i + 1, N_DEV)
        left = lax.rem(i - 1 + N_DEV, N_DEV)

        out_ref[:, :] = jnp.dot(
            x_ref[:, :], w_ref[:, :], preferred_element_type=jnp.float32
        )

        for h in range(HOPS):
            cs_cw = lax.rem(i - h + N_DEV, N_DEV)
            cr_cw = lax.rem(i - h - 1 + N_DEV, N_DEV)
            cs_ccw = lax.rem(i + h, N_DEV)
            cr_ccw = lax.rem(i + h + 1, N_DEV)
            r_cw = pltpu.make_async_remote_copy(
                src_ref=out_ref.at[pl.ds(cs_cw * ch, ch), pl.ds(0, half)],
                dst_ref=cw_ref.at[h],
                send_sem=cw_send.at[h],
                recv_sem=cw_recv.at[h],
                device_id=(right,),
                device_id_type=pl.DeviceIdType.MESH,
            )
            r_ccw = pltpu.make_async_remote_copy(
                src_ref=out_ref.at[pl.ds(cs_ccw * ch, ch), pl.ds(half, half)],
                dst_ref=ccw_ref.at[h],
                send_sem=ccw_send.at[h],
                recv_sem=ccw_recv.at[h],
                device_id=(left,),
                device_id_type=pl.DeviceIdType.MESH,
            )
            r_cw.start()
            r_ccw.start()
            r_cw.wait()
            r_ccw.wait()
            out_ref[pl.ds(cr_cw * ch, ch), pl.ds(0, half)] += cw_ref[h]
            out_ref[pl.ds(cr_ccw * ch, ch), pl.ds(half, half)] += ccw_ref[h]

        for h in range(HOPS):
            cs_cw = lax.rem(i + 1 - h + N_DEV, N_DEV)
            cs_ccw = lax.rem(i - 1 + h + N_DEV, N_DEV)
            r_cw = pltpu.make_async_remote_copy(
                src_ref=out_ref.at[pl.ds(cs_cw * ch, ch), pl.ds(0, half)],
                dst_ref=out_ref.at[pl.ds(cs_cw * ch, ch), pl.ds(0, half)],
                send_sem=ag_cw_send.at[h],
                recv_sem=ag_cw_recv.at[h],
                device_id=(right,),
                device_id_type=pl.DeviceIdType.MESH,
            )
            r_ccw = pltpu.make_async_remote_copy(
                src_ref=out_ref.at[pl.ds(cs_ccw * ch, ch), pl.ds(half, half)],
                dst_ref=out_ref.at[pl.ds(cs_ccw * ch, ch), pl.ds(half, half)],
                send_sem=ag_ccw_send.at[h],
                recv_sem=ag_ccw_recv.at[h],
                device_id=(left,),
                device_id_type=pl.DeviceIdType.MESH,
            )
            r_cw.start()
            r_ccw.start()
            r_cw.wait()
            r_ccw.wait()

    return pl.pallas_call(
        body,
        out_shape=jax.ShapeDtypeStruct((m, n), jnp.float32),
        in_specs=[
            pl.BlockSpec(memory_space=pltpu.VMEM),
            pl.BlockSpec(memory_space=pltpu.VMEM),
        ],
        out_specs=pl.BlockSpec(memory_space=pltpu.VMEM),
        scratch_shapes=[
            pltpu.VMEM((HOPS, ch, half), jnp.float32),
            pltpu.VMEM((HOPS, ch, half), jnp.float32),
            pltpu.SemaphoreType.DMA((HOPS,)),
            pltpu.SemaphoreType.DMA((HOPS,)),
            pltpu.SemaphoreType.DMA((HOPS,)),
            pltpu.SemaphoreType.DMA((HOPS,)),
            pltpu.SemaphoreType.DMA((HOPS,)),
            pltpu.SemaphoreType.DMA((HOPS,)),
            pltpu.SemaphoreType.DMA((HOPS,)),
            pltpu.SemaphoreType.DMA((HOPS,)),
        ],
    )(x, w_mat)


# device time: 136291 ns/iter; 2.6446x vs baseline; 1.5251x over previous
import jax
import jax.numpy as jnp
from jax import lax
from jax.experimental import pallas as pl
from jax.experimental.pallas import tpu as pltpu

N_DEV = 8
MASKS = {"x": 1, "y": 3, "z": 4}
ORDERS = (("x", "y", "z"), ("y", "z", "x"), ("z", "x", "y"))
SEL_A = ((3, 2, 4), (2, 4, 1), (4, 3, 1))
N_STREAMS = 3
N_STEPS = 3


def _bit(a, i):
    if a == 1:
        return i & 1
    if a == 2:
        return (i >> 1) & 1
    if a == 3:
        return (i ^ (i >> 1)) & 1
    return (i >> 2) & 1


def kernel(x, w_mat):
    m, k_shard = x.shape
    _, n = w_mat.shape

    h0 = ((m // 3) // 16) * 16 + 16
    bands = [(0, h0), (h0, h0), (2 * h0, m - 2 * h0)]
    widths = [n >> (k + 1) for k in range(N_STEPS)]

    def body(
        x_ref, w_ref, out_ref,
        rs_buf0, rs_buf1, rs_buf2,
        rs_send, rs_recv, ag_send, ag_recv,
    ):
        i = lax.axis_index("i")
        rs_bufs = [rs_buf0, rs_buf1, rs_buf2]

        out_ref[:, :] = jnp.dot(
            x_ref[:, :], w_ref[:, :], preferred_element_type=jnp.float32
        )

        offs = [0, 0, 0]
        for k in range(N_STEPS):
            w = widths[k]
            rdmas = []
            meta = []
            for s in range(N_STREAMS):
                dim = ORDERS[s][k]
                b = _bit(SEL_A[s][k], i)
                p = i ^ MASKS[dim]
                r0, h = bands[s]
                send_off = offs[s] + (1 - b) * w
                keep_off = offs[s] + b * w
                r = pltpu.make_async_remote_copy(
                    src_ref=out_ref.at[pl.ds(r0, h), pl.ds(send_off, w)],
                    dst_ref=rs_bufs[k].at[s, pl.ds(0, h)],
                    send_sem=rs_send.at[k, s],
                    recv_sem=rs_recv.at[k, s],
                    device_id=(p,),
                    device_id_type=pl.DeviceIdType.MESH,
                )
                r.start()
                rdmas.append(r)
                meta.append((r0, h, keep_off))
                offs[s] = keep_off
            for s in range(N_STREAMS):
                rdmas[s].wait()
                r0, h, keep_off = meta[s]
                out_ref[pl.ds(r0, h), pl.ds(keep_off, w)] += rs_bufs[k][
                    s, pl.ds(0, h)
                ]

        for k in reversed(range(N_STEPS)):
            w = widths[k]
            rdmas = []
            for s in range(N_STREAMS):
                dim = ORDERS[s][k]
                b = _bit(SEL_A[s][k], i)
                p = i ^ MASKS[dim]
                r0, h = bands[s]
                o = offs[s]
                r = pltpu.make_async_remote_copy(
                    src_ref=out_ref.at[pl.ds(r0, h), pl.ds(o, w)],
                    dst_ref=out_ref.at[pl.ds(r0, h), pl.ds(o, w)],
                    send_sem=ag_send.at[k, s],
                    recv_sem=ag_recv.at[k, s],
                    device_id=(p,),
                    device_id_type=pl.DeviceIdType.MESH,
                )
                r.start()
                rdmas.append(r)
                offs[s] = o - b * w
            for s in range(N_STREAMS):
                rdmas[s].wait()

    return pl.pallas_call(
        body,
        out_shape=jax.ShapeDtypeStruct((m, n), jnp.float32),
        in_specs=[
            pl.BlockSpec(memory_space=pltpu.VMEM),
            pl.BlockSpec(memory_space=pltpu.VMEM),
        ],
        out_specs=pl.BlockSpec(memory_space=pltpu.VMEM),
        scratch_shapes=[
            pltpu.VMEM((N_STREAMS, bands[0][1], widths[0]), jnp.float32),
            pltpu.VMEM((N_STREAMS, bands[0][1], widths[1]), jnp.float32),
            pltpu.VMEM((N_STREAMS, bands[0][1], widths[2]), jnp.float32),
            pltpu.SemaphoreType.DMA((N_STEPS, N_STREAMS)),
            pltpu.SemaphoreType.DMA((N_STEPS, N_STREAMS)),
            pltpu.SemaphoreType.DMA((N_STEPS, N_STREAMS)),
            pltpu.SemaphoreType.DMA((N_STEPS, N_STREAMS)),
        ],
    )(x, w_mat)


# device time: 132667 ns/iter; 2.7168x vs baseline; 1.0273x over previous
import jax
import jax.numpy as jnp
from jax import lax
from jax.experimental import pallas as pl
from jax.experimental.pallas import tpu as pltpu

N_DEV = 8
MASKS = {"x": 1, "y": 3, "z": 4}
ORDERS = (("x", "y", "z"), ("y", "z", "x"), ("z", "x", "y"))
SEL_A = ((3, 2, 4), (2, 4, 1), (4, 3, 1))
N_STREAMS = 3
N_STEPS = 3
N_SUB = 2


def _bit(a, i):
    if a == 1:
        return i & 1
    if a == 2:
        return (i >> 1) & 1
    if a == 3:
        return (i ^ (i >> 1)) & 1
    return (i >> 2) & 1


def kernel(x, w_mat):
    m, k_shard = x.shape
    _, n = w_mat.shape

    h0 = ((m // 3) // 16) * 16 + 16
    bands = [(0, h0), (h0, h0), (2 * h0, m - 2 * h0)]
    widths = [n >> (k + 1) for k in range(N_STEPS)]

    def body(
        x_ref, w_ref, out_ref,
        rs_buf0, rs_buf1, rs_buf2,
        rs_send, rs_recv, ag_send, ag_recv,
    ):
        i = lax.axis_index("i")
        rs_bufs = [rs_buf0, rs_buf1, rs_buf2]

        bits = [[_bit(SEL_A[s][k], i) for k in range(N_STEPS)]
                for s in range(N_STREAMS)]
        parts = [[i ^ MASKS[ORDERS[s][k]] for k in range(N_STEPS)]
                 for s in range(N_STREAMS)]
        offs = [[0] * (N_STEPS + 1) for _ in range(N_STREAMS)]
        for s in range(N_STREAMS):
            for k in range(N_STEPS):
                offs[s][k + 1] = offs[s][k] + bits[s][k] * widths[k]

        def rs_rdma(s, k, c):
            r0, h = bands[s]
            h2 = h // N_SUB
            w = widths[k]
            send_off = offs[s][k] + (1 - bits[s][k]) * w
            return pltpu.make_async_remote_copy(
                src_ref=out_ref.at[pl.ds(r0 + c * h2, h2), pl.ds(send_off, w)],
                dst_ref=rs_bufs[k].at[s, pl.ds(c * h2, h2)],
                send_sem=rs_send.at[k, s, c],
                recv_sem=rs_recv.at[k, s, c],
                device_id=(parts[s][k],),
                device_id_type=pl.DeviceIdType.MESH,
            )

        def rs_add(s, k, c):
            r0, h = bands[s]
            h2 = h // N_SUB
            w = widths[k]
            keep_off = offs[s][k] + bits[s][k] * w
            out_ref[pl.ds(r0 + c * h2, h2), pl.ds(keep_off, w)] += rs_bufs[k][
                s, pl.ds(c * h2, h2)
            ]

        def ag_rdma(s, k):
            r0, h = bands[s]
            w = widths[k]
            o = offs[s][k + 1]
            return pltpu.make_async_remote_copy(
                src_ref=out_ref.at[pl.ds(r0, h), pl.ds(o, w)],
                dst_ref=out_ref.at[pl.ds(r0, h), pl.ds(o, w)],
                send_sem=ag_send.at[k, s],
                recv_sem=ag_recv.at[k, s],
                device_id=(parts[s][k],),
                device_id_type=pl.DeviceIdType.MESH,
            )

        rs_inflight = {}
        ag_inflight = {}

        for s in range(N_STREAMS):
            r0, h = bands[s]
            out_ref[pl.ds(r0, h), :] = jnp.dot(
                x_ref[pl.ds(r0, h), :], w_ref[:, :],
                preferred_element_type=jnp.float32,
            )
            for c in range(N_SUB):
                r = rs_rdma(s, 0, c)
                r.start()
                rs_inflight[(s, 0, c)] = r

        for k in range(N_STEPS):
            for s in range(N_STREAMS):
                for c in range(N_SUB):
                    rs_inflight[(s, k, c)].wait()
                    rs_add(s, k, c)
                    if k + 1 < N_STEPS:
                        r = rs_rdma(s, k + 1, c)
                        r.start()
                        rs_inflight[(s, k + 1, c)] = r
                if k + 1 == N_STEPS:
                    r = ag_rdma(s, N_STEPS - 1)
                    r.start()
                    ag_inflight[(s, N_STEPS - 1)] = r

        for k in reversed(range(N_STEPS)):
            for s in range(N_STREAMS):
                ag_inflight[(s, k)].wait()
                if k > 0:
                    r = ag_rdma(s, k - 1)
                    r.start()
                    ag_inflight[(s, k - 1)] = r

    return pl.pallas_call(
        body,
        out_shape=jax.ShapeDtypeStruct((m, n), jnp.float32),
        in_specs=[
            pl.BlockSpec(memory_space=pltpu.VMEM),
            pl.BlockSpec(memory_space=pltpu.VMEM),
        ],
        out_specs=pl.BlockSpec(memory_space=pltpu.VMEM),
        scratch_shapes=[
            pltpu.VMEM((N_STREAMS, bands[0][1], widths[0]), jnp.float32),
            pltpu.VMEM((N_STREAMS, bands[0][1], widths[1]), jnp.float32),
            pltpu.VMEM((N_STREAMS, bands[0][1], widths[2]), jnp.float32),
            pltpu.SemaphoreType.DMA((N_STEPS, N_STREAMS, N_SUB)),
            pltpu.SemaphoreType.DMA((N_STEPS, N_STREAMS, N_SUB)),
            pltpu.SemaphoreType.DMA((N_STEPS, N_STREAMS)),
            pltpu.SemaphoreType.DMA((N_STEPS, N_STREAMS)),
        ],
    )(x, w_mat)


# device time: 131413 ns/iter; 2.7427x vs baseline; 1.0095x over previous
import jax
import jax.numpy as jnp
from jax import lax
from jax.experimental import pallas as pl
from jax.experimental.pallas import tpu as pltpu

N_DEV = 8
MASKS = {"x": 1, "y": 3, "z": 4}
ORDERS = (("x", "y", "z"), ("y", "z", "x"), ("z", "x", "y"))
SEL_A = ((3, 2, 4), (2, 4, 1), (4, 3, 1))
N_STREAMS = 3
N_STEPS = 3
N_SUB = 2


def _bit(a, i):
    if a == 1:
        return i & 1
    if a == 2:
        return (i >> 1) & 1
    if a == 3:
        return (i ^ (i >> 1)) & 1
    return (i >> 2) & 1


def kernel(x, w_mat):
    m, k_shard = x.shape
    _, n = w_mat.shape

    h0 = ((m // 3) // 16) * 16 + 16
    bands = [(0, h0), (h0, h0), (2 * h0, m - 2 * h0)]
    widths = [n >> (k + 1) for k in range(N_STEPS)]

    def body(
        x_ref, w_ref, out_ref,
        rs_buf0, rs_buf1, rs_buf2,
        rs_send, rs_recv, ag_send, ag_recv,
    ):
        i = lax.axis_index("i")
        rs_bufs = [rs_buf0, rs_buf1, rs_buf2]

        bits = [[_bit(SEL_A[s][k], i) for k in range(N_STEPS)]
                for s in range(N_STREAMS)]
        parts = [[i ^ MASKS[ORDERS[s][k]] for k in range(N_STEPS)]
                 for s in range(N_STREAMS)]
        offs = [[0] * (N_STEPS + 1) for _ in range(N_STREAMS)]
        for s in range(N_STREAMS):
            for k in range(N_STEPS):
                offs[s][k + 1] = offs[s][k] + bits[s][k] * widths[k]

        def rs_rdma(s, k, c):
            r0, h = bands[s]
            h2 = h // N_SUB
            w = widths[k]
            send_off = offs[s][k] + (1 - bits[s][k]) * w
            return pltpu.make_async_remote_copy(
                src_ref=out_ref.at[pl.ds(r0 + c * h2, h2), pl.ds(send_off, w)],
                dst_ref=rs_bufs[k].at[s, pl.ds(c * h2, h2)],
                send_sem=rs_send.at[k, s, c],
                recv_sem=rs_recv.at[k, s, c],
                device_id=(parts[s][k],),
                device_id_type=pl.DeviceIdType.MESH,
            )

        def rs_add(s, k, c):
            r0, h = bands[s]
            h2 = h // N_SUB
            w = widths[k]
            keep_off = offs[s][k] + bits[s][k] * w
            out_ref[pl.ds(r0 + c * h2, h2), pl.ds(keep_off, w)] += rs_bufs[k][
                s, pl.ds(c * h2, h2)
            ]

        def ag_rdma(s, k, c):
            r0, h = bands[s]
            h2 = h // N_SUB
            w = widths[k]
            o = offs[s][k + 1]
            return pltpu.make_async_remote_copy(
                src_ref=out_ref.at[pl.ds(r0 + c * h2, h2), pl.ds(o, w)],
                dst_ref=out_ref.at[pl.ds(r0 + c * h2, h2), pl.ds(o, w)],
                send_sem=ag_send.at[k, s, c],
                recv_sem=ag_recv.at[k, s, c],
                device_id=(parts[s][k],),
                device_id_type=pl.DeviceIdType.MESH,
            )

        rs_inflight = {}
        ag_inflight = {}

        for s in range(N_STREAMS):
            r0, h = bands[s]
            out_ref[pl.ds(r0, h), :] = jnp.dot(
                x_ref[pl.ds(r0, h), :], w_ref[:, :],
                preferred_element_type=jnp.float32,
            )
            for c in range(N_SUB):
                r = rs_rdma(s, 0, c)
                r.start()
                rs_inflight[(s, 0, c)] = r

        for k in range(N_STEPS):
            for s in range(N_STREAMS):
                for c in range(N_SUB):
                    rs_inflight[(s, k, c)].wait()
                    rs_add(s, k, c)
                    if k + 1 < N_STEPS:
                        r = rs_rdma(s, k + 1, c)
                        r.start()
                        rs_inflight[(s, k + 1, c)] = r
                    else:
                        r = ag_rdma(s, N_STEPS - 1, c)
                        r.start()
                        ag_inflight[(s, N_STEPS - 1, c)] = r

        for k in reversed(range(N_STEPS)):
            for s in range(N_STREAMS):
                for c in range(N_SUB):
                    ag_inflight[(s, k, c)].wait()
                    if k > 0:
                        r = ag_rdma(s, k - 1, c)
                        r.start()
                        ag_inflight[(s, k - 1, c)] = r

    return pl.pallas_call(
        body,
        out_shape=jax.ShapeDtypeStruct((m, n), jnp.float32),
        in_specs=[
            pl.BlockSpec(memory_space=pltpu.VMEM),
            pl.BlockSpec(memory_space=pltpu.VMEM),
        ],
        out_specs=pl.BlockSpec(memory_space=pltpu.VMEM),
        scratch_shapes=[
            pltpu.VMEM((N_STREAMS, bands[0][1], widths[0]), jnp.float32),
            pltpu.VMEM((N_STREAMS, bands[0][1], widths[1]), jnp.float32),
            pltpu.VMEM((N_STREAMS, bands[0][1], widths[2]), jnp.float32),
            pltpu.SemaphoreType.DMA((N_STEPS, N_STREAMS, N_SUB)),
            pltpu.SemaphoreType.DMA((N_STEPS, N_STREAMS, N_SUB)),
            pltpu.SemaphoreType.DMA((N_STEPS, N_STREAMS, N_SUB)),
            pltpu.SemaphoreType.DMA((N_STEPS, N_STREAMS, N_SUB)),
        ],
    )(x, w_mat)


# device time: 131006 ns/iter; 2.7512x vs baseline; 1.0031x over previous
import jax
import jax.numpy as jnp
from jax import lax
from jax.experimental import pallas as pl
from jax.experimental.pallas import tpu as pltpu

N_DEV = 8
MASKS = {"x": 1, "y": 3, "z": 4}
ORDERS = (("x", "y", "z"), ("y", "z", "x"), ("z", "x", "y"))
SEL_A = ((3, 2, 4), (2, 4, 1), (4, 3, 1))
N_STREAMS = 3
N_STEPS = 3
N_SUB = 2


def _bit(a, i):
    if a == 1:
        return i & 1
    if a == 2:
        return (i >> 1) & 1
    if a == 3:
        return (i ^ (i >> 1)) & 1
    return (i >> 2) & 1


def kernel(x, w_mat):
    m, k_shard = x.shape
    _, n = w_mat.shape

    h0 = ((m // 3) // 16) * 16 + 16
    bands = [(0, h0), (h0, h0), (2 * h0, m - 2 * h0)]
    widths = [n >> (k + 1) for k in range(N_STEPS)]

    def body(
        x_ref, w_ref, out_ref,
        rs_buf0, rs_buf1, rs_buf2,
        rs_send, rs_recv, ag_send, ag_recv,
    ):
        i = lax.axis_index("i")
        rs_bufs = [rs_buf0, rs_buf1, rs_buf2]

        bits = [[_bit(SEL_A[s][k], i) for k in range(N_STEPS)]
                for s in range(N_STREAMS)]
        parts = [[i ^ MASKS[ORDERS[s][k]] for k in range(N_STEPS)]
                 for s in range(N_STREAMS)]
        offs = [[0] * (N_STEPS + 1) for _ in range(N_STREAMS)]
        for s in range(N_STREAMS):
            for k in range(N_STEPS):
                offs[s][k + 1] = offs[s][k] + bits[s][k] * widths[k]

        def rs_rdma(s, k, c):
            r0, h = bands[s]
            h2 = h // N_SUB
            w = widths[k]
            send_off = offs[s][k] + (1 - bits[s][k]) * w
            return pltpu.make_async_remote_copy(
                src_ref=out_ref.at[pl.ds(r0 + c * h2, h2), pl.ds(send_off, w)],
                dst_ref=rs_bufs[k].at[s, pl.ds(c * h2, h2)],
                send_sem=rs_send.at[k, s, c],
                recv_sem=rs_recv.at[k, s, c],
                device_id=(parts[s][k],),
                device_id_type=pl.DeviceIdType.MESH,
            )

        def rs_add(s, k, c):
            r0, h = bands[s]
            h2 = h // N_SUB
            w = widths[k]
            keep_off = offs[s][k] + bits[s][k] * w
            out_ref[pl.ds(r0 + c * h2, h2), pl.ds(keep_off, w)] += rs_bufs[k][
                s, pl.ds(c * h2, h2)
            ]

        def ag_rdma(s, k, c):
            r0, h = bands[s]
            h2 = h // N_SUB
            w = widths[k]
            o = offs[s][k + 1]
            return pltpu.make_async_remote_copy(
                src_ref=out_ref.at[pl.ds(r0 + c * h2, h2), pl.ds(o, w)],
                dst_ref=out_ref.at[pl.ds(r0 + c * h2, h2), pl.ds(o, w)],
                send_sem=ag_send.at[k, s, c],
                recv_sem=ag_recv.at[k, s, c],
                device_id=(parts[s][k],),
                device_id_type=pl.DeviceIdType.MESH,
            )

        rs_inflight = {}
        ag_inflight = {}

        w0 = widths[0]
        for s in range(N_STREAMS):
            r0, h = bands[s]
            send_off = offs[s][0] + (1 - bits[s][0]) * w0
            out_ref[pl.ds(r0, h), pl.ds(send_off, w0)] = jnp.dot(
                x_ref[pl.ds(r0, h), :], w_ref[:, pl.ds(send_off, w0)],
                preferred_element_type=jnp.float32,
            )
            for c in range(N_SUB):
                r = rs_rdma(s, 0, c)
                r.start()
                rs_inflight[(s, 0, c)] = r
        for s in range(N_STREAMS):
            r0, h = bands[s]
            keep_off = offs[s][0] + bits[s][0] * w0
            out_ref[pl.ds(r0, h), pl.ds(keep_off, w0)] = jnp.dot(
                x_ref[pl.ds(r0, h), :], w_ref[:, pl.ds(keep_off, w0)],
                preferred_element_type=jnp.float32,
            )

        for k in range(N_STEPS):
            for s in range(N_STREAMS):
                for c in range(N_SUB):
                    rs_inflight[(s, k, c)].wait()
                    rs_add(s, k, c)
                    if k + 1 < N_STEPS:
                        r = rs_rdma(s, k + 1, c)
                        r.start()
                        rs_inflight[(s, k + 1, c)] = r
                    else:
                        r = ag_rdma(s, N_STEPS - 1, c)
                        r.start()
                        ag_inflight[(s, N_STEPS - 1, c)] = r

        for k in reversed(range(N_STEPS)):
            for s in range(N_STREAMS):
                for c in range(N_SUB):
                    ag_inflight[(s, k, c)].wait()
                    if k > 0:
                        r = ag_rdma(s, k - 1, c)
                        r.start()
                        ag_inflight[(s, k - 1, c)] = r

    return pl.pallas_call(
        body,
        out_shape=jax.ShapeDtypeStruct((m, n), jnp.float32),
        in_specs=[
            pl.BlockSpec(memory_space=pltpu.VMEM),
            pl.BlockSpec(memory_space=pltpu.VMEM),
        ],
        out_specs=pl.BlockSpec(memory_space=pltpu.VMEM),
        scratch_shapes=[
            pltpu.VMEM((N_STREAMS, bands[0][1], widths[0]), jnp.float32),
            pltpu.VMEM((N_STREAMS, bands[0][1], widths[1]), jnp.float32),
            pltpu.VMEM((N_STREAMS, bands[0][1], widths[2]), jnp.float32),
            pltpu.SemaphoreType.DMA((N_STEPS, N_STREAMS, N_SUB)),
            pltpu.SemaphoreType.DMA((N_STEPS, N_STREAMS, N_SUB)),
            pltpu.SemaphoreType.DMA((N_STEPS, N_STREAMS, N_SUB)),
            pltpu.SemaphoreType.DMA((N_STEPS, N_STREAMS, N_SUB)),
        ],
    )(x, w_mat)


# device time: 125849 ns/iter; 2.8640x vs baseline; 1.0410x over previous
import jax
import jax.numpy as jnp
from jax import lax
from jax.experimental import pallas as pl
from jax.experimental.pallas import tpu as pltpu

N_DEV = 8
MASKS = {"x": 1, "y": 3, "z": 4}
ORDERS = (("x", "y", "z"), ("y", "z", "x"), ("z", "x", "y"))
SEL_A = ((3, 2, 4), (2, 4, 1), (4, 3, 1))
N_STREAMS = 3
N_STEPS = 3
N_SUB = 2


def _bit(a, i):
    if a == 1:
        return i & 1
    if a == 2:
        return (i >> 1) & 1
    if a == 3:
        return (i ^ (i >> 1)) & 1
    return (i >> 2) & 1


def kernel(x, w_mat):
    m, k_shard = x.shape
    _, n = w_mat.shape

    h0 = ((m // 3) // 16) * 16 + 16
    bands = [(0, h0), (h0, h0), (2 * h0, m - 2 * h0)]
    widths = [n >> (k + 1) for k in range(N_STEPS)]

    def body(
        x_ref, w_ref, out_ref,
        rs_buf0, rs_buf1, rs_buf2,
        rs_send, rs_recv, ag_send, ag_recv,
    ):
        i = lax.axis_index("i")
        rs_bufs = [rs_buf0, rs_buf1, rs_buf2]

        bits = [[_bit(SEL_A[s][k], i) for k in range(N_STEPS)]
                for s in range(N_STREAMS)]
        parts = [[i ^ MASKS[ORDERS[s][k]] for k in range(N_STEPS)]
                 for s in range(N_STREAMS)]
        offs = [[0] * (N_STEPS + 1) for _ in range(N_STREAMS)]
        for s in range(N_STREAMS):
            for k in range(N_STEPS):
                offs[s][k + 1] = offs[s][k] + bits[s][k] * widths[k]

        def rs_rdma(s, k, c):
            r0, h = bands[s]
            h2 = h // N_SUB
            w = widths[k]
            send_off = offs[s][k] + (1 - bits[s][k]) * w
            return pltpu.make_async_remote_copy(
                src_ref=out_ref.at[pl.ds(r0 + c * h2, h2), pl.ds(send_off, w)],
                dst_ref=rs_bufs[k].at[s, pl.ds(c * h2, h2)],
                send_sem=rs_send.at[k, s, c],
                recv_sem=rs_recv.at[k, s, c],
                device_id=(parts[s][k],),
                device_id_type=pl.DeviceIdType.MESH,
            )

        def rs_add(s, k, c):
            r0, h = bands[s]
            h2 = h // N_SUB
            w = widths[k]
            keep_off = offs[s][k] + bits[s][k] * w
            out_ref[pl.ds(r0 + c * h2, h2), pl.ds(keep_off, w)] += rs_bufs[k][
                s, pl.ds(c * h2, h2)
            ]

        def ag_rdma(s, k, c):
            r0, h = bands[s]
            h2 = h // N_SUB
            w = widths[k]
            o = offs[s][k + 1]
            return pltpu.make_async_remote_copy(
                src_ref=out_ref.at[pl.ds(r0 + c * h2, h2), pl.ds(o, w)],
                dst_ref=out_ref.at[pl.ds(r0 + c * h2, h2), pl.ds(o, w)],
                send_sem=ag_send.at[k, s, c],
                recv_sem=ag_recv.at[k, s, c],
                device_id=(parts[s][k],),
                device_id_type=pl.DeviceIdType.MESH,
            )

        rs_inflight = {}
        ag_inflight = {}

        barrier_sem = pltpu.get_barrier_semaphore()
        for mask in (1, 3, 4):
            pl.semaphore_signal(
                barrier_sem, inc=1,
                device_id=(i ^ mask,), device_id_type=pl.DeviceIdType.MESH,
            )

        w0 = widths[0]
        for s in range(N_STREAMS):
            r0, h = bands[s]
            send_off = offs[s][0] + (1 - bits[s][0]) * w0
            out_ref[pl.ds(r0, h), pl.ds(send_off, w0)] = jnp.dot(
                x_ref[pl.ds(r0, h), :], w_ref[:, pl.ds(send_off, w0)],
                preferred_element_type=jnp.float32,
            )
            if s == 0:
                pl.semaphore_wait(barrier_sem, 3)
            for c in range(N_SUB):
                r = rs_rdma(s, 0, c)
                r.start()
                rs_inflight[(s, 0, c)] = r
        for s in range(N_STREAMS):
            r0, h = bands[s]
            keep_off = offs[s][0] + bits[s][0] * w0
            out_ref[pl.ds(r0, h), pl.ds(keep_off, w0)] = jnp.dot(
                x_ref[pl.ds(r0, h), :], w_ref[:, pl.ds(keep_off, w0)],
                preferred_element_type=jnp.float32,
            )

        for k in range(N_STEPS):
            for s in range(N_STREAMS):
                for c in range(N_SUB):
                    rs_inflight[(s, k, c)].wait()
                    rs_add(s, k, c)
                    if k + 1 < N_STEPS:
                        r = rs_rdma(s, k + 1, c)
                        r.start()
                        rs_inflight[(s, k + 1, c)] = r
                    else:
                        r = ag_rdma(s, N_STEPS - 1, c)
                        r.start()
                        ag_inflight[(s, N_STEPS - 1, c)] = r

        for k in reversed(range(N_STEPS)):
            for s in range(N_STREAMS):
                for c in range(N_SUB):
                    ag_inflight[(s, k, c)].wait()
                    if k > 0:
                        r = ag_rdma(s, k - 1, c)
                        r.start()
                        ag_inflight[(s, k - 1, c)] = r

    return pl.pallas_call(
        body,
        out_shape=jax.ShapeDtypeStruct((m, n), jnp.float32),
        in_specs=[
            pl.BlockSpec(memory_space=pltpu.VMEM),
            pl.BlockSpec(memory_space=pltpu.VMEM),
        ],
        out_specs=pl.BlockSpec(memory_space=pltpu.VMEM),
        scratch_shapes=[
            pltpu.VMEM((N_STREAMS, bands[0][1], widths[0]), jnp.float32),
            pltpu.VMEM((N_STREAMS, bands[0][1], widths[1]), jnp.float32),
            pltpu.VMEM((N_STREAMS, bands[0][1], widths[2]), jnp.float32),
            pltpu.SemaphoreType.DMA((N_STEPS, N_STREAMS, N_SUB)),
            pltpu.SemaphoreType.DMA((N_STEPS, N_STREAMS, N_SUB)),
            pltpu.SemaphoreType.DMA((N_STEPS, N_STREAMS, N_SUB)),
            pltpu.SemaphoreType.DMA((N_STEPS, N_STREAMS, N_SUB)),
        ],
        compiler_params=pltpu.CompilerParams(collective_id=0),
    )(x, w_mat)
